# baseline (speedup 1.0000x reference)
import functools

import jax
import jax.numpy as jnp
from jax import lax
from jax.experimental import pallas as pl
from jax.experimental.pallas import tpu as pltpu
from jax.experimental.pallas import tpu_sc as plsc

NUM_USERS = 1000000
NUM_ITEMS = 1000000
EMB_DIM = 32
BATCH = 16384

NC = 2
NS = 16
NW = NC * NS
B_PER_W = BATCH // NW


def _sc_gather_mul_body(uidx_hbm, iidx_hbm, utab_hbm, itab_hbm, x_hbm,
                        uidx_v, iidx_v, u_v, v_v, sem_u, sem_i):
    wid = lax.axis_index("s") * NC + lax.axis_index("c")
    base = wid * B_PER_W
    pltpu.sync_copy(uidx_hbm.at[pl.ds(base, B_PER_W)], uidx_v)
    pltpu.sync_copy(iidx_hbm.at[pl.ds(base, B_PER_W)], iidx_v)
    cp_u = pltpu.make_async_copy(utab_hbm.at[uidx_v], u_v, sem_u)
    cp_i = pltpu.make_async_copy(itab_hbm.at[iidx_v], v_v, sem_i)
    cp_u.start()
    cp_i.start()
    cp_u.wait()
    cp_i.wait()

    def mul_row(i, carry):
        a0 = u_v[i, pl.ds(0, 16)]
        b0 = v_v[i, pl.ds(0, 16)]
        u_v[i, pl.ds(0, 16)] = a0 * b0
        a1 = u_v[i, pl.ds(16, 16)]
        b1 = v_v[i, pl.ds(16, 16)]
        u_v[i, pl.ds(16, 16)] = a1 * b1
        return carry

    lax.fori_loop(0, B_PER_W, mul_row, 0)
    pltpu.sync_copy(u_v, x_hbm.at[pl.ds(base, B_PER_W)])


@jax.jit
def _sc_gather_mul(user_idx, item_idx, user_table, item_table):
    mesh = plsc.VectorSubcoreMesh(core_axis_name="c", subcore_axis_name="s",
                                  num_cores=NC, num_subcores=NS)
    f = pl.kernel(
        _sc_gather_mul_body,
        out_type=jax.ShapeDtypeStruct((BATCH, EMB_DIM), jnp.float32),
        mesh=mesh,
        scratch_types=[
            pltpu.VMEM((B_PER_W,), jnp.int32),
            pltpu.VMEM((B_PER_W,), jnp.int32),
            pltpu.VMEM((B_PER_W, EMB_DIM), jnp.float32),
            pltpu.VMEM((B_PER_W, EMB_DIM), jnp.float32),
            pltpu.SemaphoreType.DMA,
            pltpu.SemaphoreType.DMA,
        ],
        compiler_params=pltpu.CompilerParams(use_tc_tiling_on_sc=False),
    )
    return f(user_idx, item_idx, user_table, item_table)


def _tc_mlp_body(x_ref, w1t_ref, b1_ref, w2_ref, b2_ref, o_ref):
    x = x_ref[...]
    h = jnp.dot(x, w1t_ref[...], preferred_element_type=jnp.float32)
    h = jnp.maximum(h + b1_ref[...], 0.0)
    logits = jnp.sum(h * w2_ref[...], axis=1, keepdims=True) + b2_ref[0, 0]
    o_ref[...] = 1.0 / (1.0 + jnp.exp(-logits))


@jax.jit
def _tc_mlp(x, w1t, b1, w2, b2):
    return pl.pallas_call(
        _tc_mlp_body,
        out_shape=jax.ShapeDtypeStruct((BATCH, 1), jnp.float32),
    )(x, w1t, b1, w2, b2)


def kernel(user_idx, item_idx, user_table, item_table, W1, b1, W2, b2):
    x = _sc_gather_mul(user_idx, item_idx, user_table, item_table)
    o = _tc_mlp(x, W1.T, b1[None, :], W2, b2[None, :])
    return o[:, 0]
